# baseline (speedup 1.0000x reference)
import functools

import jax
import jax.numpy as jnp
from jax import lax
from jax.experimental import pallas as pl
from jax.experimental.pallas import tpu as pltpu
from jax.experimental.pallas import tpu_sc as plsc

SEQ = 8192
D = 2048
LANES = 16
NC = 2
NS = 16
NW = NC * NS
ROWS_PER_W = SEQ // NW
CHUNK = 8
NSTEPS = ROWS_PER_W // CHUNK
NPAIRS = NSTEPS // 2
STRIPS = D // LANES


def _pe_body(x_hbm, w_hbm, pos_hbm, out_hbm,
             idx_v,
             x0, g0, o0, x1, g1, o1,
             gs0, xs0, os0, gs1, xs1, os1):
    wid = lax.axis_index("s") * NC + lax.axis_index("c")
    base = wid * ROWS_PER_W

    pltpu.sync_copy(pos_hbm.at[pl.ds(base, ROWS_PER_W)], idx_v)

    def start_loads(s, x_v, g_v, gsem, xsem):
        row0 = base + s * CHUNK
        pltpu.async_copy(w_hbm.at[idx_v.at[pl.ds(s * CHUNK, CHUNK)]], g_v, gsem)
        pltpu.async_copy(x_hbm.at[pl.ds(row0, CHUNK)], x_v, xsem)

    def wait_loads(s, x_v, g_v, gsem, xsem):
        pltpu.make_async_copy(w_hbm.at[idx_v.at[pl.ds(s * CHUNK, CHUNK)]],
                              g_v, gsem).wait()
        pltpu.make_async_copy(x_hbm.at[pl.ds(base, CHUNK)], x_v, xsem).wait()

    def add_chunk(x_v, g_v, o_v):
        @plsc.parallel_loop(0, CHUNK, 1, unroll=2)
        def _row(r):
            for c in range(STRIPS):
                sl = pl.ds(c * LANES, LANES)
                o_v[r, sl] = x_v[r, sl] + g_v[r, sl]

    def start_store(s, o_v, osem):
        row0 = base + s * CHUNK
        pltpu.async_copy(o_v, out_hbm.at[pl.ds(row0, CHUNK)], osem)

    def wait_store(o_v, osem):
        pltpu.make_async_copy(o_v, out_hbm.at[pl.ds(base, CHUNK)], osem).wait()

    start_loads(0, x0, g0, gs0, xs0)
    start_loads(1, x1, g1, gs1, xs1)

    def pair(p, carry):
        s0 = 2 * p
        s1 = s0 + 1

        wait_loads(s0, x0, g0, gs0, xs0)

        @pl.when(p > 0)
        def _():
            wait_store(o0, os0)

        add_chunk(x0, g0, o0)
        start_store(s0, o0, os0)

        @pl.when(p < NPAIRS - 1)
        def _():
            start_loads(s0 + 2, x0, g0, gs0, xs0)

        wait_loads(s1, x1, g1, gs1, xs1)

        @pl.when(p > 0)
        def _():
            wait_store(o1, os1)

        add_chunk(x1, g1, o1)
        start_store(s1, o1, os1)

        @pl.when(p < NPAIRS - 1)
        def _():
            start_loads(s1 + 2, x1, g1, gs1, xs1)

        return carry

    lax.fori_loop(0, NPAIRS, pair, 0)

    wait_store(o0, os0)
    wait_store(o1, os1)


@jax.jit
def kernel(x, W, pos):
    mesh = plsc.VectorSubcoreMesh(core_axis_name="c", subcore_axis_name="s")
    f = pl.kernel(
        _pe_body,
        mesh=mesh,
        out_type=jax.ShapeDtypeStruct((SEQ, D), jnp.float32),
        scratch_types=[
            pltpu.VMEM((ROWS_PER_W,), jnp.int32),
            pltpu.VMEM((CHUNK, D), jnp.float32),
            pltpu.VMEM((CHUNK, D), jnp.float32),
            pltpu.VMEM((CHUNK, D), jnp.float32),
            pltpu.VMEM((CHUNK, D), jnp.float32),
            pltpu.VMEM((CHUNK, D), jnp.float32),
            pltpu.VMEM((CHUNK, D), jnp.float32),
            pltpu.SemaphoreType.DMA,
            pltpu.SemaphoreType.DMA,
            pltpu.SemaphoreType.DMA,
            pltpu.SemaphoreType.DMA,
            pltpu.SemaphoreType.DMA,
            pltpu.SemaphoreType.DMA,
        ],
    )
    return f(x, W, pos)


# speedup vs baseline: 1.1108x; 1.1108x over previous
import functools

import jax
import jax.numpy as jnp
from jax import lax
from jax.experimental import pallas as pl
from jax.experimental.pallas import tpu as pltpu
from jax.experimental.pallas import tpu_sc as plsc

SEQ = 8192
D = 2048
LANES = 16
NC = 2
NS = 16
NW = NC * NS
ROWS_PER_W = SEQ // NW
CHUNK = 8
NSTEPS = ROWS_PER_W // CHUNK
NOBUF = 4
NGBUF = 2
NGROUPS = NSTEPS // NOBUF
STRIPS = D // LANES


def _pe_body(x_hbm, w_hbm, pos_hbm, out_hbm, idx_v, *refs):
    o_v = refs[0:NOBUF]
    g_v = refs[NOBUF:NOBUF + NGBUF]
    xs = refs[NOBUF + NGBUF:2 * NOBUF + NGBUF]
    os_ = refs[2 * NOBUF + NGBUF:3 * NOBUF + NGBUF]
    gs = refs[3 * NOBUF + NGBUF:3 * NOBUF + 2 * NGBUF]

    wid = lax.axis_index("s") * NC + lax.axis_index("c")
    base = wid * ROWS_PER_W

    pltpu.sync_copy(pos_hbm.at[pl.ds(base, ROWS_PER_W)], idx_v)

    def start_xload(s, bo):
        pltpu.async_copy(x_hbm.at[pl.ds(base + s * CHUNK, CHUNK)],
                         o_v[bo], xs[bo])

    def wait_xload(bo):
        pltpu.make_async_copy(x_hbm.at[pl.ds(base, CHUNK)], o_v[bo],
                              xs[bo]).wait()

    def start_gather(s, bg):
        pltpu.async_copy(w_hbm.at[idx_v.at[pl.ds(s * CHUNK, CHUNK)]],
                         g_v[bg], gs[bg])

    def wait_gather(bg):
        pltpu.make_async_copy(w_hbm.at[idx_v.at[pl.ds(0, CHUNK)]],
                              g_v[bg], gs[bg]).wait()

    def accum_chunk(bg, bo):
        @plsc.parallel_loop(0, CHUNK, 1)
        def _row(r):
            for c in range(STRIPS):
                sl = pl.ds(c * LANES, LANES)
                plsc.addupdate(o_v[bo].at[r, sl], g_v[bg][r, sl])

    def start_store(s, bo):
        pltpu.async_copy(o_v[bo], out_hbm.at[pl.ds(base + s * CHUNK, CHUNK)],
                         os_[bo])

    def wait_store(bo):
        pltpu.make_async_copy(o_v[bo], out_hbm.at[pl.ds(base, CHUNK)],
                              os_[bo]).wait()

    start_xload(0, 0)
    start_gather(0, 0)
    start_xload(1, 1)
    start_gather(1, 1)

    def group(p, carry):
        for j in range(NOBUF):
            s = NOBUF * p + j
            bo = j
            bg = j % NGBUF
            b2 = (j + 2) % NOBUF

            wait_gather(bg)
            wait_xload(bo)
            accum_chunk(bg, bo)
            start_store(s, bo)

            @pl.when(s < NSTEPS - 2)
            def _():
                start_gather(s + 2, bg)

                @pl.when(s >= 2)
                def _():
                    wait_store(b2)
                start_xload(s + 2, b2)

        return carry

    lax.fori_loop(0, NGROUPS, group, 0)

    for bo in range(NOBUF):
        wait_store(bo)


@jax.jit
def kernel(x, W, pos):
    mesh = plsc.VectorSubcoreMesh(core_axis_name="c", subcore_axis_name="s")
    f = pl.kernel(
        _pe_body,
        mesh=mesh,
        out_type=jax.ShapeDtypeStruct((SEQ, D), jnp.float32),
        scratch_types=(
            [pltpu.VMEM((ROWS_PER_W,), jnp.int32)]
            + [pltpu.VMEM((CHUNK, D), jnp.float32)
               for _ in range(NOBUF + NGBUF)]
            + [pltpu.SemaphoreType.DMA for _ in range(2 * NOBUF + NGBUF)]
        ),
    )
    return f(x, W, pos)


# speedup vs baseline: 1.1761x; 1.0588x over previous
import jax
import jax.numpy as jnp
from jax import lax
from jax.experimental import pallas as pl
from jax.experimental.pallas import tpu as pltpu
from jax.experimental.pallas import tpu_sc as plsc

SEQ = 8192
D = 2048
LANES = 16
NC = 2
NS = 16
NW = NC * NS
ROWS_PER_W = SEQ // NW
CHUNK = 8
NSTEPS = ROWS_PER_W // CHUNK
NPAIRS = NSTEPS // 2
STRIPS = D // LANES


def _pe_body(x_hbm, w_hbm, pos_hbm, out_hbm,
             idx_v,
             x0, g0, o0, x1, g1, o1,
             gs0, xs0, os0, gs1, xs1, os1):
    wid = lax.axis_index("s") * NC + lax.axis_index("c")
    base = wid * ROWS_PER_W

    pltpu.sync_copy(pos_hbm.at[pl.ds(base, ROWS_PER_W)], idx_v)

    def start_loads(s, x_v, g_v, gsem, xsem):
        row0 = base + s * CHUNK
        pltpu.async_copy(w_hbm.at[idx_v.at[pl.ds(s * CHUNK, CHUNK)]], g_v, gsem)
        pltpu.async_copy(x_hbm.at[pl.ds(row0, CHUNK)], x_v, xsem)

    def wait_loads(s, x_v, g_v, gsem, xsem):
        pltpu.make_async_copy(w_hbm.at[idx_v.at[pl.ds(s * CHUNK, CHUNK)]],
                              g_v, gsem).wait()
        pltpu.make_async_copy(x_hbm.at[pl.ds(base, CHUNK)], x_v, xsem).wait()

    def add_chunk(x_v, g_v, o_v):
        @plsc.parallel_loop(0, CHUNK, 1)
        def _row(r):
            for c in range(STRIPS):
                sl = pl.ds(c * LANES, LANES)
                o_v[r, sl] = x_v[r, sl] + g_v[r, sl]

    def start_store(s, o_v, osem):
        row0 = base + s * CHUNK
        pltpu.async_copy(o_v, out_hbm.at[pl.ds(row0, CHUNK)], osem)

    def wait_store(o_v, osem):
        pltpu.make_async_copy(o_v, out_hbm.at[pl.ds(base, CHUNK)], osem).wait()

    start_loads(0, x0, g0, gs0, xs0)
    start_loads(1, x1, g1, gs1, xs1)

    def pair(p, carry):
        s0 = 2 * p
        s1 = s0 + 1

        wait_loads(s0, x0, g0, gs0, xs0)

        @pl.when(p > 0)
        def _():
            wait_store(o0, os0)

        add_chunk(x0, g0, o0)
        start_store(s0, o0, os0)

        @pl.when(p < NPAIRS - 1)
        def _():
            start_loads(s0 + 2, x0, g0, gs0, xs0)

        wait_loads(s1, x1, g1, gs1, xs1)

        @pl.when(p > 0)
        def _():
            wait_store(o1, os1)

        add_chunk(x1, g1, o1)
        start_store(s1, o1, os1)

        @pl.when(p < NPAIRS - 1)
        def _():
            start_loads(s1 + 2, x1, g1, gs1, xs1)

        return carry

    lax.fori_loop(0, NPAIRS, pair, 0)

    wait_store(o0, os0)
    wait_store(o1, os1)


@jax.jit
def kernel(x, W, pos):
    mesh = plsc.VectorSubcoreMesh(core_axis_name="c", subcore_axis_name="s")
    f = pl.kernel(
        _pe_body,
        mesh=mesh,
        out_type=jax.ShapeDtypeStruct((SEQ, D), jnp.float32),
        scratch_types=[
            pltpu.VMEM((ROWS_PER_W,), jnp.int32),
            pltpu.VMEM((CHUNK, D), jnp.float32),
            pltpu.VMEM((CHUNK, D), jnp.float32),
            pltpu.VMEM((CHUNK, D), jnp.float32),
            pltpu.VMEM((CHUNK, D), jnp.float32),
            pltpu.VMEM((CHUNK, D), jnp.float32),
            pltpu.VMEM((CHUNK, D), jnp.float32),
            pltpu.SemaphoreType.DMA,
            pltpu.SemaphoreType.DMA,
            pltpu.SemaphoreType.DMA,
            pltpu.SemaphoreType.DMA,
            pltpu.SemaphoreType.DMA,
            pltpu.SemaphoreType.DMA,
        ],
    )
    return f(x, W, pos)


# speedup vs baseline: 1.1775x; 1.0012x over previous
import jax
import jax.numpy as jnp
from jax import lax
from jax.experimental import pallas as pl
from jax.experimental.pallas import tpu as pltpu
from jax.experimental.pallas import tpu_sc as plsc

SEQ = 8192
D = 2048
LANES = 16
NC = 2
NS = 16
NW = NC * NS
ROWS_PER_W = SEQ // NW
CHUNK = 8
NSTEPS = ROWS_PER_W // CHUNK
NPAIRS = NSTEPS // 2
STRIPS = D // LANES


def _pe_body(x_hbm, w_hbm, pos_hbm, out_hbm,
             idx_v,
             x0, g0, o0, x1, g1, o1,
             gs0, xs0, os0, gs1, xs1, os1):
    wid = lax.axis_index("c") * NS + lax.axis_index("s")
    base = wid * ROWS_PER_W

    pltpu.sync_copy(pos_hbm.at[pl.ds(base, ROWS_PER_W)], idx_v)

    def start_loads(s, x_v, g_v, gsem, xsem):
        row0 = base + s * CHUNK
        pltpu.async_copy(w_hbm.at[idx_v.at[pl.ds(s * CHUNK, CHUNK)]], g_v, gsem)
        pltpu.async_copy(x_hbm.at[pl.ds(row0, CHUNK)], x_v, xsem)

    def wait_loads(s, x_v, g_v, gsem, xsem):
        pltpu.make_async_copy(w_hbm.at[idx_v.at[pl.ds(s * CHUNK, CHUNK)]],
                              g_v, gsem).wait()
        pltpu.make_async_copy(x_hbm.at[pl.ds(base, CHUNK)], x_v, xsem).wait()

    def add_chunk(x_v, g_v, o_v):
        @plsc.parallel_loop(0, CHUNK, 1)
        def _row(r):
            for c in range(STRIPS):
                sl = pl.ds(c * LANES, LANES)
                o_v[r, sl] = x_v[r, sl] + g_v[r, sl]

    def start_store(s, o_v, osem):
        row0 = base + s * CHUNK
        pltpu.async_copy(o_v, out_hbm.at[pl.ds(row0, CHUNK)], osem)

    def wait_store(o_v, osem):
        pltpu.make_async_copy(o_v, out_hbm.at[pl.ds(base, CHUNK)], osem).wait()

    start_loads(0, x0, g0, gs0, xs0)
    start_loads(1, x1, g1, gs1, xs1)

    def pair(p, carry):
        s0 = 2 * p
        s1 = s0 + 1

        wait_loads(s0, x0, g0, gs0, xs0)

        @pl.when(p > 0)
        def _():
            wait_store(o0, os0)

        add_chunk(x0, g0, o0)
        start_store(s0, o0, os0)

        @pl.when(p < NPAIRS - 1)
        def _():
            start_loads(s0 + 2, x0, g0, gs0, xs0)

        wait_loads(s1, x1, g1, gs1, xs1)

        @pl.when(p > 0)
        def _():
            wait_store(o1, os1)

        add_chunk(x1, g1, o1)
        start_store(s1, o1, os1)

        @pl.when(p < NPAIRS - 1)
        def _():
            start_loads(s1 + 2, x1, g1, gs1, xs1)

        return carry

    lax.fori_loop(0, NPAIRS, pair, 0)

    wait_store(o0, os0)
    wait_store(o1, os1)


@jax.jit
def kernel(x, W, pos):
    mesh = plsc.VectorSubcoreMesh(core_axis_name="c", subcore_axis_name="s")
    f = pl.kernel(
        _pe_body,
        mesh=mesh,
        out_type=jax.ShapeDtypeStruct((SEQ, D), jnp.float32),
        scratch_types=[
            pltpu.VMEM((ROWS_PER_W,), jnp.int32),
            pltpu.VMEM((CHUNK, D), jnp.float32),
            pltpu.VMEM((CHUNK, D), jnp.float32),
            pltpu.VMEM((CHUNK, D), jnp.float32),
            pltpu.VMEM((CHUNK, D), jnp.float32),
            pltpu.VMEM((CHUNK, D), jnp.float32),
            pltpu.VMEM((CHUNK, D), jnp.float32),
            pltpu.SemaphoreType.DMA,
            pltpu.SemaphoreType.DMA,
            pltpu.SemaphoreType.DMA,
            pltpu.SemaphoreType.DMA,
            pltpu.SemaphoreType.DMA,
            pltpu.SemaphoreType.DMA,
        ],
    )
    return f(x, W, pos)


# speedup vs baseline: 1.1829x; 1.0046x over previous
import jax
import jax.numpy as jnp
from jax import lax
from jax.experimental import pallas as pl
from jax.experimental.pallas import tpu as pltpu
from jax.experimental.pallas import tpu_sc as plsc

SEQ = 8192
D = 2048
LANES = 16
NC = 2
NS = 16
NW = NC * NS
ROWS_PER_W = SEQ // NW
CHUNK = 8
NSTEPS = ROWS_PER_W // CHUNK
NPAIRS = NSTEPS // 2
STRIPS = D // LANES


def _pe_body(x_hbm, w_hbm, pos_hbm, out_hbm,
             idx_v,
             x0, g0, o0, x1, g1, o1,
             gs0, xs0, os0, gs1, xs1, os1):
    wid = lax.axis_index("c") * NS + lax.axis_index("s")
    base = wid * ROWS_PER_W

    pltpu.sync_copy(pos_hbm.at[pl.ds(base, ROWS_PER_W)], idx_v)

    def start_loads(s, x_v, g_v, gsem, xsem):
        row0 = base + s * CHUNK
        pltpu.async_copy(x_hbm.at[pl.ds(row0, CHUNK)], x_v, xsem)
        pltpu.async_copy(w_hbm.at[idx_v.at[pl.ds(s * CHUNK, CHUNK)]], g_v, gsem)

    def wait_loads(s, x_v, g_v, gsem, xsem):
        pltpu.make_async_copy(w_hbm.at[idx_v.at[pl.ds(s * CHUNK, CHUNK)]],
                              g_v, gsem).wait()
        pltpu.make_async_copy(x_hbm.at[pl.ds(base, CHUNK)], x_v, xsem).wait()

    def add_chunk(x_v, g_v, o_v):
        @plsc.parallel_loop(0, CHUNK, 1)
        def _row(r):
            for c in range(STRIPS):
                sl = pl.ds(c * LANES, LANES)
                o_v[r, sl] = x_v[r, sl] + g_v[r, sl]

    def start_store(s, o_v, osem):
        row0 = base + s * CHUNK
        pltpu.async_copy(o_v, out_hbm.at[pl.ds(row0, CHUNK)], osem)

    def wait_store(o_v, osem):
        pltpu.make_async_copy(o_v, out_hbm.at[pl.ds(base, CHUNK)], osem).wait()

    start_loads(0, x0, g0, gs0, xs0)
    start_loads(1, x1, g1, gs1, xs1)

    def pair(p, carry):
        s0 = 2 * p
        s1 = s0 + 1

        wait_loads(s0, x0, g0, gs0, xs0)

        @pl.when(p > 0)
        def _():
            wait_store(o0, os0)

        add_chunk(x0, g0, o0)
        start_store(s0, o0, os0)

        @pl.when(p < NPAIRS - 1)
        def _():
            start_loads(s0 + 2, x0, g0, gs0, xs0)

        wait_loads(s1, x1, g1, gs1, xs1)

        @pl.when(p > 0)
        def _():
            wait_store(o1, os1)

        add_chunk(x1, g1, o1)
        start_store(s1, o1, os1)

        @pl.when(p < NPAIRS - 1)
        def _():
            start_loads(s1 + 2, x1, g1, gs1, xs1)

        return carry

    lax.fori_loop(0, NPAIRS, pair, 0)

    wait_store(o0, os0)
    wait_store(o1, os1)


@jax.jit
def kernel(x, W, pos):
    mesh = plsc.VectorSubcoreMesh(core_axis_name="c", subcore_axis_name="s")
    f = pl.kernel(
        _pe_body,
        mesh=mesh,
        out_type=jax.ShapeDtypeStruct((SEQ, D), jnp.float32),
        scratch_types=[
            pltpu.VMEM((ROWS_PER_W,), jnp.int32),
            pltpu.VMEM((CHUNK, D), jnp.float32),
            pltpu.VMEM((CHUNK, D), jnp.float32),
            pltpu.VMEM((CHUNK, D), jnp.float32),
            pltpu.VMEM((CHUNK, D), jnp.float32),
            pltpu.VMEM((CHUNK, D), jnp.float32),
            pltpu.VMEM((CHUNK, D), jnp.float32),
            pltpu.SemaphoreType.DMA,
            pltpu.SemaphoreType.DMA,
            pltpu.SemaphoreType.DMA,
            pltpu.SemaphoreType.DMA,
            pltpu.SemaphoreType.DMA,
            pltpu.SemaphoreType.DMA,
        ],
    )
    return f(x, W, pos)
